# baseline (speedup 1.0000x reference)
import functools

import jax
import jax.numpy as jnp
from jax import lax
from jax.experimental import pallas as pl
from jax.experimental.pallas import tpu as pltpu
from jax.experimental.pallas import tpu_sc as plsc

_P = 1024
_E = 768
_H = _E // 2
_M = 16
_NC, _NS, _L = 2, 16, 16
_NW = _NC * _NS
_PPW = _P // _NW


def _pos_build_sc(idx_hbm, sp_hbm, im_hbm, pos_hbm, idx_v, sp_v, row_v, out_v, sem):
    wid = lax.axis_index("s") * _NC + lax.axis_index("c")
    base = wid * _PPW
    pltpu.sync_copy(idx_hbm, idx_v)
    pltpu.async_copy(im_hbm.at[idx_v], row_v, sem).wait()
    pltpu.sync_copy(sp_hbm.at[pl.ds(base, _PPW)], sp_v)
    rvals = [row_v[0, pl.ds(_L * j, _L)] for j in range(_H // _L)]
    for p in range(_PPW):
        for j in range(_H // _L):
            out_v[p, pl.ds(_L * j, _L)] = sp_v[p, pl.ds(_L * j, _L)]
        for j in range(_H // _L):
            out_v[p, pl.ds(_H + _L * j, _L)] = rvals[j]
    pltpu.sync_copy(out_v, pos_hbm.at[pl.ds(base, _PPW)])


def _build_pos(idx, spatial2d, image2d):
    mesh = plsc.VectorSubcoreMesh(core_axis_name="c", subcore_axis_name="s")
    return pl.kernel(
        _pos_build_sc,
        mesh=mesh,
        out_type=jax.ShapeDtypeStruct((_P, _E), jnp.float32),
        scratch_types=[
            pltpu.VMEM((1,), jnp.int32),
            pltpu.VMEM((_PPW, _H), jnp.float32),
            pltpu.VMEM((1, _H), jnp.float32),
            pltpu.VMEM((_PPW, _E), jnp.float32),
            pltpu.SemaphoreType.DMA,
        ],
    )(idx, spatial2d, image2d)


def _add_body(x_ref, pos_ref, o_ref):
    o_ref[...] = x_ref[...] + pos_ref[...]


def kernel(x, image_idx, spatial_pos_embed, image_pos_embed):
    B, P, E = x.shape
    idx = jnp.asarray(image_idx, jnp.int32).reshape(1)
    pos = _build_pos(idx, spatial_pos_embed.reshape(_P, _H),
                     image_pos_embed.reshape(_M, _H))
    bb = 4
    return pl.pallas_call(
        _add_body,
        grid=(B // bb,),
        in_specs=[
            pl.BlockSpec((bb, P, E), lambda b: (b, 0, 0)),
            pl.BlockSpec((1, P, E), lambda b: (0, 0, 0)),
        ],
        out_specs=pl.BlockSpec((bb, P, E), lambda b: (b, 0, 0)),
        out_shape=jax.ShapeDtypeStruct((B, P, E), x.dtype),
        compiler_params=pltpu.CompilerParams(
            dimension_semantics=("arbitrary",),
        ),
    )(x, pos.reshape(1, P, E))
